# Initial kernel scaffold; baseline (speedup 1.0000x reference)
import jax
import jax.numpy as jnp
from jax.experimental import pallas as pl


def kernel(z, codebook):
    raise NotImplementedError("write your pallas kernel here")


# speedup vs baseline: 1.0147x; 1.0147x over previous
import functools

import jax
import jax.numpy as jnp
from jax import lax
from jax.experimental import pallas as pl
from jax.experimental.pallas import tpu as pltpu
from jax.experimental.pallas import tpu_sc as plsc

_NUM_CODEBOOK = 8192
_DIM = 256
_BETA = 0.25

_TT = 256
_PREC = jax.lax.Precision.DEFAULT


def _dist_argmin_body(z_ref, z2_ref, cbt_ref, c2_ref, idx_ref):
    m = lax.dot_general(
        z_ref[...], cbt_ref[...], (((1,), (0,)), ((), ())),
        precision=_PREC, preferred_element_type=jnp.float32)
    dist = (z2_ref[...] + c2_ref[...]) - 2.0 * m
    minv = jnp.min(dist, axis=1, keepdims=True)
    iota = lax.broadcasted_iota(jnp.int32, dist.shape, 1)
    idx = jnp.min(jnp.where(dist == minv, iota, jnp.int32(_NUM_CODEBOOK)),
                  axis=1)
    idx_ref[...] = idx[:, None]


def _dist_argmin(z_flat, z2, cbt, c2):
    n = z_flat.shape[0]
    return pl.pallas_call(
        _dist_argmin_body,
        grid=(n // _TT,),
        in_specs=[
            pl.BlockSpec((_TT, _DIM), lambda i: (i, 0)),
            pl.BlockSpec((_TT, 1), lambda i: (i, 0)),
            pl.BlockSpec((_DIM, _NUM_CODEBOOK), lambda i: (0, 0)),
            pl.BlockSpec((1, _NUM_CODEBOOK), lambda i: (0, 0)),
        ],
        out_specs=pl.BlockSpec((_TT, 1), lambda i: (i, 0)),
        out_shape=jax.ShapeDtypeStruct((n, 1), jnp.int32),
    )(z_flat, z2, cbt, c2)


def _sc_gather(codebook, idx):
    info = plsc.get_sparse_core_info()
    nw = info.num_cores * info.num_subcores
    b = idx.shape[0]
    b_per_w = b // nw
    mesh = plsc.VectorSubcoreMesh(core_axis_name="c", subcore_axis_name="s")

    @functools.partial(
        pl.kernel, mesh=mesh,
        out_type=jax.ShapeDtypeStruct((b, _DIM), codebook.dtype),
        scratch_types=[
            pltpu.VMEM((b_per_w,), jnp.int32),
            pltpu.VMEM((b_per_w, _DIM), codebook.dtype),
            pltpu.SemaphoreType.DMA,
        ],
    )
    def k(table_hbm, idx_hbm, out_hbm, idx_v, rows_v, sem):
        wid = lax.axis_index("s") * info.num_cores + lax.axis_index("c")
        base = wid * b_per_w
        pltpu.sync_copy(idx_hbm.at[pl.ds(base, b_per_w)], idx_v)
        pltpu.async_copy(table_hbm.at[idx_v], rows_v, sem).wait()
        pltpu.sync_copy(rows_v, out_hbm.at[pl.ds(base, b_per_w)])

    return k(codebook, idx)


def _st_loss_body(z_ref, zq_ref, out_ref, loss_ref):
    i = pl.program_id(0)
    z = z_ref[...]
    d = zq_ref[...] - z
    out_ref[...] = z + d

    @pl.when(i == 0)
    def _():
        loss_ref[...] = jnp.zeros_like(loss_ref)

    loss_ref[...] += jnp.sum(d * d).reshape(1, 1)


def _st_loss(z_flat, zq_flat):
    n = z_flat.shape[0]
    return pl.pallas_call(
        _st_loss_body,
        grid=(n // _TT,),
        in_specs=[
            pl.BlockSpec((_TT, _DIM), lambda i: (i, 0)),
            pl.BlockSpec((_TT, _DIM), lambda i: (i, 0)),
        ],
        out_specs=[
            pl.BlockSpec((_TT, _DIM), lambda i: (i, 0)),
            pl.BlockSpec((1, 1), lambda i: (0, 0)),
        ],
        out_shape=[
            jax.ShapeDtypeStruct((n, _DIM), jnp.float32),
            jax.ShapeDtypeStruct((1, 1), jnp.float32),
        ],
    )(z_flat, zq_flat)


def kernel(z, codebook):
    B, C, H, W = z.shape
    z_perm = jnp.transpose(z, (0, 2, 3, 1))
    z_flat = z_perm.reshape(-1, C)
    z2 = jnp.sum(z_flat ** 2, axis=1, keepdims=True)
    c2 = jnp.sum(codebook ** 2, axis=1).reshape(1, -1)
    cbt = codebook.T

    idx = _dist_argmin(z_flat, z2, cbt, c2)
    zq_flat = _sc_gather(codebook, idx.reshape(-1))
    zq_st_flat, loss_sum = _st_loss(z_flat, zq_flat)

    m1 = loss_sum[0, 0] / z.size
    commit_loss = m1 + _BETA * m1
    z_q_st = jnp.transpose(zq_st_flat.reshape(B, H, W, C), (0, 3, 1, 2))
    return (z_q_st, commit_loss)
